# baseline (speedup 1.0000x reference)
import jax
import jax.numpy as jnp
from jax import lax
from jax.experimental import pallas as pl
from jax.experimental.pallas import tpu as pltpu
from jax.experimental.pallas import tpu_sc as plsc

NN = 10000
EE = 320000
FD = 128
MM = 100
NCLS = 16
NEG = 0.2

NROW = 10240
EPAD = 327680
EW = EPAD // 32
ECH = EW // 128
MDPAD = 10240
MDW = MDPAD // 32
RB = 1024
NBLK = NROW // RB

_mesh = plsc.VectorSubcoreMesh(core_axis_name="c", subcore_axis_name="s")


def _lrelu(t):
    return jnp.where(t >= 0, t, t * NEG)


def _dot(a, b):
    return jnp.dot(a, b, preferred_element_type=jnp.float32,
                   precision=lax.Precision.HIGHEST)


def _logsoftmax(z):
    m = jnp.max(z, axis=1, keepdims=True)
    zs = z - m
    return zs - jnp.log(jnp.sum(jnp.exp(zs), axis=1, keepdims=True))


def _deg_body(dst_hbm, md_hbm, cnt_out, mcnt_out, dbuf, acc, mbuf, macc):
    c = lax.axis_index("c")
    s = lax.axis_index("s")
    wid = c * 16 + s
    zeros16 = jnp.zeros((16,), jnp.float32)
    ones16 = jnp.ones((16,), jnp.float32)

    def z1(i, carry):
        acc[pl.ds(i * 16, 16)] = zeros16
        return carry
    lax.fori_loop(0, NROW // 16, z1, 0)
    for i in range(8):
        macc[pl.ds(i * 16, 16)] = zeros16

    pltpu.sync_copy(dst_hbm.at[wid], dbuf)

    def cu(i, carry):
        def cu2(j, carry2):
            idx = dbuf[i, pl.ds(j * 16, 16)]
            plsc.addupdate_scatter(acc, [idx], ones16)
            return carry2
        return lax.fori_loop(0, 8, cu2, carry)
    lax.fori_loop(0, ECH, cu, 0)

    pltpu.sync_copy(md_hbm.at[pl.ds(wid * MDW, MDW)], mbuf)

    def cm(i, carry):
        idx = mbuf[pl.ds(i * 16, 16)]
        plsc.addupdate_scatter(macc, [idx], ones16)
        return carry
    lax.fori_loop(0, MDW // 16, cm, 0)

    pltpu.sync_copy(acc, cnt_out.at[c, s])
    pltpu.sync_copy(macc, mcnt_out.at[c, s])


_deg_call = pl.kernel(
    _deg_body,
    out_type=(jax.ShapeDtypeStruct((2, 16, NROW), jnp.float32),
              jax.ShapeDtypeStruct((2, 16, 128), jnp.float32)),
    mesh=_mesh,
    scratch_types=[
        pltpu.VMEM((ECH, 128), jnp.int32),
        pltpu.VMEM((NROW,), jnp.float32),
        pltpu.VMEM((MDW,), jnp.int32),
        pltpu.VMEM((128,), jnp.float32),
    ],
    compiler_params=pltpu.CompilerParams(needs_layout_passes=False),
)


def _agg_body(xs_hbm, src_hbm, dst_hbm, out_hbm, idx_s, idx_d, rows, acc,
              sem):
    c = lax.axis_index("c")
    s = lax.axis_index("s")
    wid = c * 16 + s
    zeros16 = jnp.zeros((16,), jnp.float32)

    def zr(i, carry):
        def zc(j, carry2):
            rows[i, pl.ds(j * 16, 16)] = zeros16
            return carry2
        return lax.fori_loop(0, 8, zc, carry)
    lax.fori_loop(0, 128, zr, 0)
    for k in range(5):
        pltpu.sync_copy(rows, acc.at[pl.ds(s * 640 + k * 128, 128)])
    plsc.subcore_barrier()

    def step(k, carry):
        base = wid * EW + k * 128
        pltpu.sync_copy(src_hbm.at[pl.ds(base, 128)], idx_s)
        pltpu.sync_copy(dst_hbm.at[pl.ds(base, 128)], idx_d)
        pltpu.async_copy(xs_hbm.at[idx_s], rows, sem).wait()
        pltpu.sync_copy(rows, acc.at[idx_d], add=True)
        return carry
    lax.fori_loop(0, ECH, step, 0)

    plsc.subcore_barrier()
    for k in range(5):
        pltpu.sync_copy(acc.at[pl.ds(s * 640 + k * 128, 128)],
                        out_hbm.at[c, s, k])


_agg_call = pl.kernel(
    _agg_body,
    out_type=jax.ShapeDtypeStruct((2, 16, 5, 128, FD), jnp.float32),
    mesh=_mesh,
    scratch_types=[
        pltpu.VMEM((128,), jnp.int32),
        pltpu.VMEM((128,), jnp.int32),
        pltpu.VMEM((128, FD), jnp.float32),
        pltpu.VMEM_SHARED((NROW, FD), jnp.float32),
        pltpu.SemaphoreType.DMA,
    ],
)


def _tca_body(x_ref, cnt_ref, wl_ref, bl_ref, wc0_ref,
              xs1_ref, xw1_ref, dinv_ref):
    cnt = jnp.sum(cnt_ref[...], axis=0)
    dinv = lax.rsqrt(cnt + 1.0)
    h = _lrelu(_dot(x_ref[...], wl_ref[...]) + bl_ref[...][None, :])
    xw1 = _dot(h, wc0_ref[...])
    xs1_ref[...] = xw1 * dinv[:, None]
    xw1_ref[...] = xw1
    dinv_ref[...] = dinv


def _tca(xp, cnt32, W_lin, b_lin, Wc0):
    return pl.pallas_call(
        _tca_body,
        grid=(NBLK,),
        in_specs=[
            pl.BlockSpec((RB, FD), lambda i: (i, 0)),
            pl.BlockSpec((32, RB), lambda i: (0, i)),
            pl.BlockSpec((FD, FD), lambda i: (0, 0)),
            pl.BlockSpec((FD,), lambda i: (0,)),
            pl.BlockSpec((FD, FD), lambda i: (0, 0)),
        ],
        out_specs=[
            pl.BlockSpec((RB, FD), lambda i: (i, 0)),
            pl.BlockSpec((RB, FD), lambda i: (i, 0)),
            pl.BlockSpec((RB,), lambda i: (i,)),
        ],
        out_shape=[
            jax.ShapeDtypeStruct((NROW, FD), jnp.float32),
            jax.ShapeDtypeStruct((NROW, FD), jnp.float32),
            jax.ShapeDtypeStruct((NROW,), jnp.float32),
        ],
    )(xp, cnt32, W_lin, b_lin, Wc0)


def _tcc_body(p0_ref, p1_ref, xw1_ref, dinv_ref, bc0_ref, wc1_ref,
              xs2_ref, xw2_ref):
    dinv = dinv_ref[...]
    agg = p0_ref[...] + p1_ref[...]
    out1 = _lrelu(agg * dinv[:, None] + xw1_ref[...] * (dinv * dinv)[:, None]
                  + bc0_ref[...][None, :])
    xw2 = _dot(out1, wc1_ref[...])
    xs2_ref[...] = xw2 * dinv[:, None]
    xw2_ref[...] = xw2


def _tcc(p0, p1, xw1, dinv, bc0, Wc1):
    return pl.pallas_call(
        _tcc_body,
        grid=(NBLK,),
        in_specs=[
            pl.BlockSpec((RB, FD), lambda i: (i, 0)),
            pl.BlockSpec((RB, FD), lambda i: (i, 0)),
            pl.BlockSpec((RB, FD), lambda i: (i, 0)),
            pl.BlockSpec((RB,), lambda i: (i,)),
            pl.BlockSpec((FD,), lambda i: (0,)),
            pl.BlockSpec((FD, FD), lambda i: (0, 0)),
        ],
        out_specs=[
            pl.BlockSpec((RB, FD), lambda i: (i, 0)),
            pl.BlockSpec((RB, FD), lambda i: (i, 0)),
        ],
        out_shape=[
            jax.ShapeDtypeStruct((NROW, FD), jnp.float32),
            jax.ShapeDtypeStruct((NROW, FD), jnp.float32),
        ],
    )(p0, p1, xw1, dinv, bc0, Wc1)


def _te1_body(p0_ref, p1_ref, xw2_ref, dinv_ref, bc1_ref, wg_ref, md_ref,
              bg_ref, wcls_ref, bcls_ref, ls_ref, seg_ref):
    i = pl.program_id(0)
    dinv = dinv_ref[...]
    h2 = _lrelu((p0_ref[...] + p1_ref[...]) * dinv[:, None]
                + xw2_ref[...] * (dinv * dinv)[:, None]
                + bc1_ref[...][None, :])
    xwm = _dot(h2, wg_ref[...])
    ob = _lrelu(xwm + bg_ref[...][None, :])
    logits = _dot(ob, wcls_ref[...]) + bcls_ref[...][None, :]
    ls_ref[...] = _logsoftmax(logits)

    iota = lax.broadcasted_iota(jnp.int32, (128, RB), 0)
    oh = jnp.where(iota == md_ref[...][None, :], 1.0, 0.0).astype(jnp.float32)
    segc = _dot(oh, xwm)

    @pl.when(i == 0)
    def _init():
        seg_ref[...] = segc

    @pl.when(i != 0)
    def _accum():
        seg_ref[...] = seg_ref[...] + segc


def _te1(p0, p1, xw2, dinv, bc1, Wg, mdp, bg, Wcls, bcls):
    return pl.pallas_call(
        _te1_body,
        grid=(NBLK,),
        in_specs=[
            pl.BlockSpec((RB, FD), lambda i: (i, 0)),
            pl.BlockSpec((RB, FD), lambda i: (i, 0)),
            pl.BlockSpec((RB, FD), lambda i: (i, 0)),
            pl.BlockSpec((RB,), lambda i: (i,)),
            pl.BlockSpec((FD,), lambda i: (0,)),
            pl.BlockSpec((FD, FD), lambda i: (0, 0)),
            pl.BlockSpec((RB,), lambda i: (i,)),
            pl.BlockSpec((FD,), lambda i: (0,)),
            pl.BlockSpec((FD, NCLS), lambda i: (0, 0)),
            pl.BlockSpec((NCLS,), lambda i: (0,)),
        ],
        out_specs=[
            pl.BlockSpec((RB, NCLS), lambda i: (i, 0)),
            pl.BlockSpec((128, FD), lambda i: (0, 0)),
        ],
        out_shape=[
            jax.ShapeDtypeStruct((NROW, NCLS), jnp.float32),
            jax.ShapeDtypeStruct((128, FD), jnp.float32),
        ],
    )(p0, p1, xw2, dinv, bc1, Wg, mdp, bg, Wcls, bcls)


def _te2_body(seg_ref, mcnt_ref, mx_ref, wm_ref, bm_ref, wg_ref, bg_ref,
              wcls_ref, bcls_ref, out_ref):
    mcnt = jnp.sum(mcnt_ref[...], axis=0)
    dinvm = lax.rsqrt(mcnt + 1.0)
    mh = _lrelu(_dot(mx_ref[...], wm_ref[...]) + bm_ref[...][None, :])
    xwm = _dot(mh, wg_ref[...])
    seg104 = seg_ref[0:104, :]
    d104 = dinvm[0:104]
    hm = _lrelu(seg104 * d104[:, None] + xwm * (d104 * d104)[:, None]
                + bg_ref[...][None, :])
    logits = _dot(hm, wcls_ref[...]) + bcls_ref[...][None, :]
    out_ref[...] = _logsoftmax(logits)


def _te2(seg, mcnt32, mxp, W_meta, b_meta, Wg, bg, Wcls, bcls):
    return pl.pallas_call(
        _te2_body,
        out_shape=jax.ShapeDtypeStruct((104, NCLS), jnp.float32),
    )(seg, mcnt32, mxp, W_meta, b_meta, Wg, bg, Wcls, bcls)


def kernel(x, meta_x, edge_index, meta_edge_index, W_lin, b_lin, W_meta,
           b_meta, Wc0, bc0, Wc1, bc1, Wg, bg, Wcls, bcls):
    src = edge_index[0].astype(jnp.int32)
    dst = edge_index[1].astype(jnp.int32)
    srcp = jnp.concatenate([src, jnp.zeros((EPAD - EE,), jnp.int32)])
    dstp = jnp.concatenate([dst, jnp.full((EPAD - EE,), NN, jnp.int32)])
    dstp_deg = dstp.reshape(32, ECH, 128)
    mdp = jnp.concatenate([meta_edge_index[1].astype(jnp.int32) - NN,
                           jnp.full((MDPAD - NN,), MM, jnp.int32)])
    mxp = jnp.concatenate([meta_x, jnp.zeros((104 - MM, FD), meta_x.dtype)],
                          axis=0)

    cnt_parts, mcnt_parts = _deg_call(dstp_deg, mdp)
    cnt32 = cnt_parts.reshape(32, NROW)
    mcnt32 = mcnt_parts.reshape(32, 128)

    xs1, xw1, dinv = _tca(x, cnt32, W_lin, b_lin, Wc0)
    agg1 = _agg_call(xs1, srcp, dstp).reshape(2, NROW, FD)
    xs2, xw2 = _tcc(agg1[0], agg1[1], xw1, dinv, bc0, Wc1)
    agg2 = _agg_call(xs2, srcp, dstp).reshape(2, NROW, FD)
    base_ls, seg = _te1(agg2[0], agg2[1], xw2, dinv, bc1, Wg, mdp, bg,
                        Wcls, bcls)
    meta_ls = _te2(seg, mcnt32, mxp, W_meta, b_meta, Wg, bg, Wcls, bcls)
    return jnp.concatenate([base_ls[:NN], meta_ls[:MM]], axis=0)


# speedup vs baseline: 1.0327x; 1.0327x over previous
import jax
import jax.numpy as jnp
from jax import lax
from jax.experimental import pallas as pl
from jax.experimental.pallas import tpu as pltpu
from jax.experimental.pallas import tpu_sc as plsc

NN = 10000
EE = 320000
FD = 128
MM = 100
NCLS = 16
NEG = 0.2

NROW = 10240
EPAD = 327680
EW = EPAD // 32
ECH = EW // 128
MDPAD = 10240
MDW = MDPAD // 32
RB = 1024
NBLK = NROW // RB

_mesh = plsc.VectorSubcoreMesh(core_axis_name="c", subcore_axis_name="s")


def _lrelu(t):
    return jnp.where(t >= 0, t, t * NEG)


def _dot(a, b):
    return jnp.dot(a, b, preferred_element_type=jnp.float32,
                   precision=lax.Precision.HIGHEST)


def _logsoftmax(z):
    m = jnp.max(z, axis=1, keepdims=True)
    zs = z - m
    return zs - jnp.log(jnp.sum(jnp.exp(zs), axis=1, keepdims=True))


def _deg_body(dst_hbm, md_hbm, cnt_out, mcnt_out, dbuf, acc, mbuf, macc):
    c = lax.axis_index("c")
    s = lax.axis_index("s")
    wid = c * 16 + s
    zeros16 = jnp.zeros((16,), jnp.float32)
    ones16 = jnp.ones((16,), jnp.float32)

    def z1(i, carry):
        acc[pl.ds(i * 16, 16)] = zeros16
        return carry
    lax.fori_loop(0, NROW // 16, z1, 0)
    for i in range(8):
        macc[pl.ds(i * 16, 16)] = zeros16

    pltpu.sync_copy(dst_hbm.at[wid], dbuf)

    def cu(i, carry):
        def cu2(j, carry2):
            idx = dbuf[i, pl.ds(j * 16, 16)]
            plsc.addupdate_scatter(acc, [idx], ones16)
            return carry2
        return lax.fori_loop(0, 8, cu2, carry)
    lax.fori_loop(0, ECH, cu, 0)

    pltpu.sync_copy(md_hbm.at[pl.ds(wid * MDW, MDW)], mbuf)

    def cm(i, carry):
        idx = mbuf[pl.ds(i * 16, 16)]
        plsc.addupdate_scatter(macc, [idx], ones16)
        return carry
    lax.fori_loop(0, MDW // 16, cm, 0)

    pltpu.sync_copy(acc, cnt_out.at[c, s])
    pltpu.sync_copy(macc, mcnt_out.at[c, s])


_deg_call = pl.kernel(
    _deg_body,
    out_type=(jax.ShapeDtypeStruct((2, 16, NROW), jnp.float32),
              jax.ShapeDtypeStruct((2, 16, 128), jnp.float32)),
    mesh=_mesh,
    scratch_types=[
        pltpu.VMEM((ECH, 128), jnp.int32),
        pltpu.VMEM((NROW,), jnp.float32),
        pltpu.VMEM((MDW,), jnp.int32),
        pltpu.VMEM((128,), jnp.float32),
    ],
    compiler_params=pltpu.CompilerParams(needs_layout_passes=False),
)


def _agg_body(xs_hbm, src_hbm, dst_hbm, out_hbm, idx_s, idx_d, rows, acc,
              sem):
    c = lax.axis_index("c")
    s = lax.axis_index("s")
    wid = c * 16 + s
    zeros16 = jnp.zeros((16,), jnp.float32)

    def zr(i, carry):
        def zc(j, carry2):
            rows[i, pl.ds(j * 16, 16)] = zeros16
            return carry2
        return lax.fori_loop(0, 8, zc, carry)
    lax.fori_loop(0, 128, zr, 0)
    for k in range(5):
        pltpu.sync_copy(rows, acc.at[pl.ds(s * 640 + k * 128, 128)])
    plsc.subcore_barrier()

    def step(k, carry):
        base = wid * EW + k * 128
        pltpu.sync_copy(src_hbm.at[pl.ds(base, 128)], idx_s)
        pltpu.sync_copy(dst_hbm.at[pl.ds(base, 128)], idx_d)
        pltpu.async_copy(xs_hbm.at[idx_s], rows, sem).wait()
        pltpu.sync_copy(rows, acc.at[idx_d], add=True)
        return carry
    lax.fori_loop(0, ECH, step, 0)

    plsc.subcore_barrier()
    for k in range(5):
        pltpu.sync_copy(acc.at[pl.ds(s * 640 + k * 128, 128)],
                        out_hbm.at[c, s, k])


_agg_call = pl.kernel(
    _agg_body,
    out_type=jax.ShapeDtypeStruct((2, 16, 5, 128, FD), jnp.float32),
    mesh=_mesh,
    scratch_types=[
        pltpu.VMEM((128,), jnp.int32),
        pltpu.VMEM((128,), jnp.int32),
        pltpu.VMEM((128, FD), jnp.float32),
        pltpu.VMEM_SHARED((NROW, FD), jnp.float32),
        pltpu.SemaphoreType.DMA,
    ],
)


def _tca_body(x_ref, cnt_ref, wl_ref, bl_ref, wc0_ref,
              xs1_ref, xw1_ref, dinv_ref):
    cnt = jnp.sum(cnt_ref[...], axis=0)
    dinv = lax.rsqrt(cnt + 1.0)
    h = _lrelu(_dot(x_ref[...], wl_ref[...]) + bl_ref[...][None, :])
    xw1 = _dot(h, wc0_ref[...])
    xs1_ref[...] = xw1 * dinv[:, None]
    xw1_ref[...] = xw1
    dinv_ref[...] = dinv


def _tca(xp, cnt32, W_lin, b_lin, Wc0):
    return pl.pallas_call(
        _tca_body,
        grid=(NBLK,),
        in_specs=[
            pl.BlockSpec((RB, FD), lambda i: (i, 0)),
            pl.BlockSpec((32, RB), lambda i: (0, i)),
            pl.BlockSpec((FD, FD), lambda i: (0, 0)),
            pl.BlockSpec((FD,), lambda i: (0,)),
            pl.BlockSpec((FD, FD), lambda i: (0, 0)),
        ],
        out_specs=[
            pl.BlockSpec((RB, FD), lambda i: (i, 0)),
            pl.BlockSpec((RB, FD), lambda i: (i, 0)),
            pl.BlockSpec((RB,), lambda i: (i,)),
        ],
        out_shape=[
            jax.ShapeDtypeStruct((NROW, FD), jnp.float32),
            jax.ShapeDtypeStruct((NROW, FD), jnp.float32),
            jax.ShapeDtypeStruct((NROW,), jnp.float32),
        ],
    )(xp, cnt32, W_lin, b_lin, Wc0)


def _tcc_body(p0_ref, p1_ref, xw1_ref, dinv_ref, bc0_ref, wc1_ref,
              xs2_ref, xw2_ref):
    dinv = dinv_ref[...]
    agg = p0_ref[...] + p1_ref[...]
    out1 = _lrelu(agg * dinv[:, None] + xw1_ref[...] * (dinv * dinv)[:, None]
                  + bc0_ref[...][None, :])
    xw2 = _dot(out1, wc1_ref[...])
    xs2_ref[...] = xw2 * dinv[:, None]
    xw2_ref[...] = xw2


def _tcc(p0, p1, xw1, dinv, bc0, Wc1):
    return pl.pallas_call(
        _tcc_body,
        grid=(NBLK,),
        in_specs=[
            pl.BlockSpec((RB, FD), lambda i: (i, 0)),
            pl.BlockSpec((RB, FD), lambda i: (i, 0)),
            pl.BlockSpec((RB, FD), lambda i: (i, 0)),
            pl.BlockSpec((RB,), lambda i: (i,)),
            pl.BlockSpec((FD,), lambda i: (0,)),
            pl.BlockSpec((FD, FD), lambda i: (0, 0)),
        ],
        out_specs=[
            pl.BlockSpec((RB, FD), lambda i: (i, 0)),
            pl.BlockSpec((RB, FD), lambda i: (i, 0)),
        ],
        out_shape=[
            jax.ShapeDtypeStruct((NROW, FD), jnp.float32),
            jax.ShapeDtypeStruct((NROW, FD), jnp.float32),
        ],
    )(p0, p1, xw1, dinv, bc0, Wc1)


def _te1_body(p0_ref, p1_ref, xw2_ref, dinv_ref, bc1_ref, wg_ref, md_ref,
              bg_ref, wcls_ref, bcls_ref, ls_ref, seg_ref):
    i = pl.program_id(0)
    dinv = dinv_ref[...]
    h2 = _lrelu((p0_ref[...] + p1_ref[...]) * dinv[:, None]
                + xw2_ref[...] * (dinv * dinv)[:, None]
                + bc1_ref[...][None, :])
    xwm = _dot(h2, wg_ref[...])
    ob = _lrelu(xwm + bg_ref[...][None, :])
    logits = _dot(ob, wcls_ref[...]) + bcls_ref[...][None, :]
    ls_ref[...] = _logsoftmax(logits)

    iota = lax.broadcasted_iota(jnp.int32, (128, RB), 0)
    oh = jnp.where(iota == md_ref[...][None, :], 1.0, 0.0).astype(jnp.float32)
    segc = _dot(oh, xwm)

    @pl.when(i == 0)
    def _init():
        seg_ref[...] = segc

    @pl.when(i != 0)
    def _accum():
        seg_ref[...] = seg_ref[...] + segc


def _te1(p0, p1, xw2, dinv, bc1, Wg, mdp, bg, Wcls, bcls):
    return pl.pallas_call(
        _te1_body,
        grid=(NBLK,),
        in_specs=[
            pl.BlockSpec((RB, FD), lambda i: (i, 0)),
            pl.BlockSpec((RB, FD), lambda i: (i, 0)),
            pl.BlockSpec((RB, FD), lambda i: (i, 0)),
            pl.BlockSpec((RB,), lambda i: (i,)),
            pl.BlockSpec((FD,), lambda i: (0,)),
            pl.BlockSpec((FD, FD), lambda i: (0, 0)),
            pl.BlockSpec((RB,), lambda i: (i,)),
            pl.BlockSpec((FD,), lambda i: (0,)),
            pl.BlockSpec((FD, NCLS), lambda i: (0, 0)),
            pl.BlockSpec((NCLS,), lambda i: (0,)),
        ],
        out_specs=[
            pl.BlockSpec((RB, NCLS), lambda i: (i, 0)),
            pl.BlockSpec((128, FD), lambda i: (0, 0)),
        ],
        out_shape=[
            jax.ShapeDtypeStruct((NROW, NCLS), jnp.float32),
            jax.ShapeDtypeStruct((128, FD), jnp.float32),
        ],
    )(p0, p1, xw2, dinv, bc1, Wg, mdp, bg, Wcls, bcls)


def _te2_body(seg_ref, mcnt_ref, mx_ref, wm_ref, bm_ref, wg_ref, bg_ref,
              wcls_ref, bcls_ref, out_ref):
    mcnt = jnp.sum(mcnt_ref[...], axis=0)
    dinvm = lax.rsqrt(mcnt + 1.0)
    mh = _lrelu(_dot(mx_ref[...], wm_ref[...]) + bm_ref[...][None, :])
    xwm = _dot(mh, wg_ref[...])
    seg104 = seg_ref[0:104, :]
    d104 = dinvm[0:104]
    hm = _lrelu(seg104 * d104[:, None] + xwm * (d104 * d104)[:, None]
                + bg_ref[...][None, :])
    logits = _dot(hm, wcls_ref[...]) + bcls_ref[...][None, :]
    out_ref[...] = _logsoftmax(logits)


def _te2(seg, mcnt32, mxp, W_meta, b_meta, Wg, bg, Wcls, bcls):
    return pl.pallas_call(
        _te2_body,
        out_shape=jax.ShapeDtypeStruct((104, NCLS), jnp.float32),
    )(seg, mcnt32, mxp, W_meta, b_meta, Wg, bg, Wcls, bcls)


def kernel(x, meta_x, edge_index, meta_edge_index, W_lin, b_lin, W_meta,
           b_meta, Wc0, bc0, Wc1, bc1, Wg, bg, Wcls, bcls):
    src = edge_index[0].astype(jnp.int32)
    dst = edge_index[1].astype(jnp.int32)
    srcp = jnp.concatenate([src, jnp.zeros((EPAD - EE,), jnp.int32)])
    dstp = jnp.concatenate([dst, jnp.full((EPAD - EE,), NN, jnp.int32)])
    dstp_deg = dstp.reshape(32, ECH, 128)
    mdp = jnp.concatenate([meta_edge_index[1].astype(jnp.int32) - NN,
                           jnp.full((MDPAD - NN,), MM, jnp.int32)])
    xp = jnp.concatenate([x, jnp.zeros((NROW - NN, FD), x.dtype)], axis=0)
    mxp = jnp.concatenate([meta_x, jnp.zeros((104 - MM, FD), meta_x.dtype)],
                          axis=0)

    cnt_parts, mcnt_parts = _deg_call(dstp_deg, mdp)
    cnt32 = cnt_parts.reshape(32, NROW)
    mcnt32 = mcnt_parts.reshape(32, 128)

    xs1, xw1, dinv = _tca(xp, cnt32, W_lin, b_lin, Wc0)
    agg1 = _agg_call(xs1, srcp, dstp).reshape(2, NROW, FD)
    xs2, xw2 = _tcc(agg1[0], agg1[1], xw1, dinv, bc0, Wc1)
    agg2 = _agg_call(xs2, srcp, dstp).reshape(2, NROW, FD)
    base_ls, seg = _te1(agg2[0], agg2[1], xw2, dinv, bc1, Wg, mdp, bg,
                        Wcls, bcls)
    meta_ls = _te2(seg, mcnt32, mxp, W_meta, b_meta, Wg, bg, Wcls, bcls)
    return jnp.concatenate([base_ls[:NN], meta_ls[:MM]], axis=0)


# speedup vs baseline: 1.4827x; 1.4358x over previous
import jax
import jax.numpy as jnp
from jax import lax
from jax.experimental import pallas as pl
from jax.experimental.pallas import tpu as pltpu
from jax.experimental.pallas import tpu_sc as plsc

NN = 10000
EE = 320000
FD = 128
MM = 100
NCLS = 16
NEG = 0.2

NROW = 10240
EPAD = 323584
EW = EPAD // 32
ECH = EW // 128
MDPAD = 10240
MDW = MDPAD // 32
RB = 1024
NBLK = NROW // RB

_mesh = plsc.VectorSubcoreMesh(core_axis_name="c", subcore_axis_name="s")


def _lrelu(t):
    return jnp.where(t >= 0, t, t * NEG)


def _dot(a, b):
    return jnp.dot(a, b, preferred_element_type=jnp.float32,
                   precision=lax.Precision.HIGHEST)


def _logsoftmax(z):
    m = jnp.max(z, axis=1, keepdims=True)
    zs = z - m
    return zs - jnp.log(jnp.sum(jnp.exp(zs), axis=1, keepdims=True))


def _deg_body(dst_hbm, md_hbm, cnt_out, mcnt_out, dbuf, acc, mbuf, macc):
    c = lax.axis_index("c")
    s = lax.axis_index("s")
    wid = c * 16 + s
    zeros16 = jnp.zeros((16,), jnp.float32)
    ones16 = jnp.ones((16,), jnp.float32)

    def z1(i, carry):
        acc[pl.ds(i * 16, 16)] = zeros16
        return carry
    lax.fori_loop(0, NROW // 16, z1, 0)
    for i in range(8):
        macc[pl.ds(i * 16, 16)] = zeros16

    pltpu.sync_copy(dst_hbm.at[pl.ds(wid * EW, EW)], dbuf)

    def cu(i, carry):
        idx = dbuf[pl.ds(i * 16, 16)]
        plsc.addupdate_scatter(acc, [idx], ones16)
        return carry
    lax.fori_loop(0, EW // 16, cu, 0)

    pltpu.sync_copy(md_hbm.at[pl.ds(wid * MDW, MDW)], mbuf)

    def cm(i, carry):
        idx = mbuf[pl.ds(i * 16, 16)]
        plsc.addupdate_scatter(macc, [idx], ones16)
        return carry
    lax.fori_loop(0, MDW // 16, cm, 0)

    pltpu.sync_copy(acc, cnt_out.at[c, s])
    pltpu.sync_copy(macc, mcnt_out.at[c, s])


_deg_call = pl.kernel(
    _deg_body,
    out_type=(jax.ShapeDtypeStruct((2, 16, NROW), jnp.float32),
              jax.ShapeDtypeStruct((2, 16, 128), jnp.float32)),
    mesh=_mesh,
    scratch_types=[
        pltpu.VMEM((EW,), jnp.int32),
        pltpu.VMEM((NROW,), jnp.float32),
        pltpu.VMEM((MDW,), jnp.int32),
        pltpu.VMEM((128,), jnp.float32),
    ],
    compiler_params=pltpu.CompilerParams(needs_layout_passes=False),
)


def _agg_body(xs_hbm, src_hbm, dst_hbm, out_hbm, idx_s, idx_d, rows, acc,
              sem):
    c = lax.axis_index("c")
    s = lax.axis_index("s")
    wid = c * 16 + s
    zeros16 = jnp.zeros((16,), jnp.float32)

    def zr(i, carry):
        def zc(j, carry2):
            rows[i, pl.ds(j * 16, 16)] = zeros16
            return carry2
        return lax.fori_loop(0, 8, zc, carry)
    lax.fori_loop(0, 128, zr, 0)
    for k in range(5):
        pltpu.sync_copy(rows, acc.at[pl.ds(s * 640 + k * 128, 128)])
    plsc.subcore_barrier()

    def step(k, carry):
        base = wid * EW + k * 128
        pltpu.sync_copy(src_hbm.at[pl.ds(base, 128)], idx_s)
        pltpu.sync_copy(dst_hbm.at[pl.ds(base, 128)], idx_d)
        pltpu.async_copy(xs_hbm.at[idx_s], rows, sem).wait()
        pltpu.sync_copy(rows, acc.at[idx_d], add=True)
        return carry
    lax.fori_loop(0, ECH, step, 0)

    plsc.subcore_barrier()
    for k in range(5):
        pltpu.sync_copy(acc.at[pl.ds(s * 640 + k * 128, 128)],
                        out_hbm.at[c, s, k])


_agg_call = pl.kernel(
    _agg_body,
    out_type=jax.ShapeDtypeStruct((2, 16, 5, 128, FD), jnp.float32),
    mesh=_mesh,
    scratch_types=[
        pltpu.VMEM((128,), jnp.int32),
        pltpu.VMEM((128,), jnp.int32),
        pltpu.VMEM((128, FD), jnp.float32),
        pltpu.VMEM_SHARED((NROW, FD), jnp.float32),
        pltpu.SemaphoreType.DMA,
    ],
)


def _tca_body(x_ref, cnt_ref, wl_ref, bl_ref, wc0_ref,
              xs1_ref, xw1_ref, dinv_ref):
    cnt = jnp.sum(cnt_ref[...], axis=0)
    dinv = lax.rsqrt(cnt + 1.0)
    h = _lrelu(_dot(x_ref[...], wl_ref[...]) + bl_ref[...][None, :])
    xw1 = _dot(h, wc0_ref[...])
    xs1_ref[...] = xw1 * dinv[:, None]
    xw1_ref[...] = xw1
    dinv_ref[...] = dinv


def _tca(xp, cnt32, W_lin, b_lin, Wc0):
    return pl.pallas_call(
        _tca_body,
        grid=(NBLK,),
        in_specs=[
            pl.BlockSpec((RB, FD), lambda i: (i, 0)),
            pl.BlockSpec((32, RB), lambda i: (0, i)),
            pl.BlockSpec((FD, FD), lambda i: (0, 0)),
            pl.BlockSpec((FD,), lambda i: (0,)),
            pl.BlockSpec((FD, FD), lambda i: (0, 0)),
        ],
        out_specs=[
            pl.BlockSpec((RB, FD), lambda i: (i, 0)),
            pl.BlockSpec((RB, FD), lambda i: (i, 0)),
            pl.BlockSpec((RB,), lambda i: (i,)),
        ],
        out_shape=[
            jax.ShapeDtypeStruct((NROW, FD), jnp.float32),
            jax.ShapeDtypeStruct((NROW, FD), jnp.float32),
            jax.ShapeDtypeStruct((NROW,), jnp.float32),
        ],
    )(xp, cnt32, W_lin, b_lin, Wc0)


def _tcc_body(p0_ref, p1_ref, xw1_ref, dinv_ref, bc0_ref, wc1_ref,
              xs2_ref, xw2_ref):
    dinv = dinv_ref[...]
    agg = p0_ref[...] + p1_ref[...]
    out1 = _lrelu(agg * dinv[:, None] + xw1_ref[...] * (dinv * dinv)[:, None]
                  + bc0_ref[...][None, :])
    xw2 = _dot(out1, wc1_ref[...])
    xs2_ref[...] = xw2 * dinv[:, None]
    xw2_ref[...] = xw2


def _tcc(p0, p1, xw1, dinv, bc0, Wc1):
    return pl.pallas_call(
        _tcc_body,
        grid=(NBLK,),
        in_specs=[
            pl.BlockSpec((RB, FD), lambda i: (i, 0)),
            pl.BlockSpec((RB, FD), lambda i: (i, 0)),
            pl.BlockSpec((RB, FD), lambda i: (i, 0)),
            pl.BlockSpec((RB,), lambda i: (i,)),
            pl.BlockSpec((FD,), lambda i: (0,)),
            pl.BlockSpec((FD, FD), lambda i: (0, 0)),
        ],
        out_specs=[
            pl.BlockSpec((RB, FD), lambda i: (i, 0)),
            pl.BlockSpec((RB, FD), lambda i: (i, 0)),
        ],
        out_shape=[
            jax.ShapeDtypeStruct((NROW, FD), jnp.float32),
            jax.ShapeDtypeStruct((NROW, FD), jnp.float32),
        ],
    )(p0, p1, xw1, dinv, bc0, Wc1)


def _te1_body(p0_ref, p1_ref, xw2_ref, dinv_ref, bc1_ref, wg_ref, md_ref,
              bg_ref, wcls_ref, bcls_ref, ls_ref, seg_ref):
    i = pl.program_id(0)
    dinv = dinv_ref[...]
    h2 = _lrelu((p0_ref[...] + p1_ref[...]) * dinv[:, None]
                + xw2_ref[...] * (dinv * dinv)[:, None]
                + bc1_ref[...][None, :])
    xwm = _dot(h2, wg_ref[...])
    ob = _lrelu(xwm + bg_ref[...][None, :])
    logits = _dot(ob, wcls_ref[...]) + bcls_ref[...][None, :]
    ls_ref[...] = _logsoftmax(logits)

    iota = lax.broadcasted_iota(jnp.int32, (128, RB), 0)
    oh = jnp.where(iota == md_ref[...][None, :], 1.0, 0.0).astype(jnp.float32)
    segc = _dot(oh, xwm)

    @pl.when(i == 0)
    def _init():
        seg_ref[...] = segc

    @pl.when(i != 0)
    def _accum():
        seg_ref[...] = seg_ref[...] + segc


def _te1(p0, p1, xw2, dinv, bc1, Wg, mdp, bg, Wcls, bcls):
    return pl.pallas_call(
        _te1_body,
        grid=(NBLK,),
        in_specs=[
            pl.BlockSpec((RB, FD), lambda i: (i, 0)),
            pl.BlockSpec((RB, FD), lambda i: (i, 0)),
            pl.BlockSpec((RB, FD), lambda i: (i, 0)),
            pl.BlockSpec((RB,), lambda i: (i,)),
            pl.BlockSpec((FD,), lambda i: (0,)),
            pl.BlockSpec((FD, FD), lambda i: (0, 0)),
            pl.BlockSpec((RB,), lambda i: (i,)),
            pl.BlockSpec((FD,), lambda i: (0,)),
            pl.BlockSpec((FD, NCLS), lambda i: (0, 0)),
            pl.BlockSpec((NCLS,), lambda i: (0,)),
        ],
        out_specs=[
            pl.BlockSpec((RB, NCLS), lambda i: (i, 0)),
            pl.BlockSpec((128, FD), lambda i: (0, 0)),
        ],
        out_shape=[
            jax.ShapeDtypeStruct((NROW, NCLS), jnp.float32),
            jax.ShapeDtypeStruct((128, FD), jnp.float32),
        ],
    )(p0, p1, xw2, dinv, bc1, Wg, mdp, bg, Wcls, bcls)


def _te2_body(seg_ref, mcnt_ref, mx_ref, wm_ref, bm_ref, wg_ref, bg_ref,
              wcls_ref, bcls_ref, out_ref):
    mcnt = jnp.sum(mcnt_ref[...], axis=0)
    dinvm = lax.rsqrt(mcnt + 1.0)
    mh = _lrelu(_dot(mx_ref[...], wm_ref[...]) + bm_ref[...][None, :])
    xwm = _dot(mh, wg_ref[...])
    seg104 = seg_ref[0:104, :]
    d104 = dinvm[0:104]
    hm = _lrelu(seg104 * d104[:, None] + xwm * (d104 * d104)[:, None]
                + bg_ref[...][None, :])
    logits = _dot(hm, wcls_ref[...]) + bcls_ref[...][None, :]
    out_ref[...] = _logsoftmax(logits)


def _te2(seg, mcnt32, mxp, W_meta, b_meta, Wg, bg, Wcls, bcls):
    return pl.pallas_call(
        _te2_body,
        out_shape=jax.ShapeDtypeStruct((104, NCLS), jnp.float32),
    )(seg, mcnt32, mxp, W_meta, b_meta, Wg, bg, Wcls, bcls)


def kernel(x, meta_x, edge_index, meta_edge_index, W_lin, b_lin, W_meta,
           b_meta, Wc0, bc0, Wc1, bc1, Wg, bg, Wcls, bcls):
    src = edge_index[0].astype(jnp.int32)
    dst = edge_index[1].astype(jnp.int32)
    srcp = jnp.concatenate([src, jnp.zeros((EPAD - EE,), jnp.int32)])
    dstp = jnp.concatenate([dst, jnp.full((EPAD - EE,), NN, jnp.int32)])
    mdp = jnp.concatenate([meta_edge_index[1].astype(jnp.int32) - NN,
                           jnp.full((MDPAD - NN,), MM, jnp.int32)])
    xp = jnp.concatenate([x, jnp.zeros((NROW - NN, FD), x.dtype)], axis=0)
    mxp = jnp.concatenate([meta_x, jnp.zeros((104 - MM, FD), meta_x.dtype)],
                          axis=0)

    cnt_parts, mcnt_parts = _deg_call(dstp, mdp)
    cnt32 = cnt_parts.reshape(32, NROW)
    mcnt32 = mcnt_parts.reshape(32, 128)

    xs1, xw1, dinv = _tca(xp, cnt32, W_lin, b_lin, Wc0)
    agg1 = _agg_call(xs1, srcp, dstp).reshape(2, NROW, FD)
    xs2, xw2 = _tcc(agg1[0], agg1[1], xw1, dinv, bc0, Wc1)
    agg2 = _agg_call(xs2, srcp, dstp).reshape(2, NROW, FD)
    base_ls, seg = _te1(agg2[0], agg2[1], xw2, dinv, bc1, Wg, mdp, bg,
                        Wcls, bcls)
    meta_ls = _te2(seg, mcnt32, mxp, W_meta, b_meta, Wg, bg, Wcls, bcls)
    return jnp.concatenate([base_ls[:NN], meta_ls[:MM]], axis=0)


# speedup vs baseline: 1.5265x; 1.0295x over previous
import jax
import jax.numpy as jnp
from jax import lax
from jax.experimental import pallas as pl
from jax.experimental.pallas import tpu as pltpu
from jax.experimental.pallas import tpu_sc as plsc

NN = 10000
EE = 320000
FD = 128
MM = 100
NCLS = 16
NEG = 0.2

NROW = 10240
EPAD = 323584
EW = EPAD // 32
ECH = EW // 128
MDPAD = 10240
MDW = MDPAD // 32
RB = 1024
NBLK = NROW // RB

_mesh = plsc.VectorSubcoreMesh(core_axis_name="c", subcore_axis_name="s")


def _lrelu(t):
    return jnp.where(t >= 0, t, t * NEG)


def _dot(a, b):
    return jnp.dot(a, b, preferred_element_type=jnp.float32)


def _logsoftmax(z):
    m = jnp.max(z, axis=1, keepdims=True)
    zs = z - m
    return zs - jnp.log(jnp.sum(jnp.exp(zs), axis=1, keepdims=True))


def _deg_body(dst_hbm, md_hbm, cnt_out, mcnt_out, dbuf, acc, mbuf, macc):
    c = lax.axis_index("c")
    s = lax.axis_index("s")
    wid = c * 16 + s
    zeros16 = jnp.zeros((16,), jnp.float32)
    ones16 = jnp.ones((16,), jnp.float32)

    def z1(i, carry):
        acc[pl.ds(i * 16, 16)] = zeros16
        return carry
    lax.fori_loop(0, NROW // 16, z1, 0)
    for i in range(8):
        macc[pl.ds(i * 16, 16)] = zeros16

    pltpu.sync_copy(dst_hbm.at[pl.ds(wid * EW, EW)], dbuf)

    def cu(i, carry):
        idx = dbuf[pl.ds(i * 16, 16)]
        plsc.addupdate_scatter(acc, [idx], ones16)
        return carry
    lax.fori_loop(0, EW // 16, cu, 0)

    pltpu.sync_copy(md_hbm.at[pl.ds(wid * MDW, MDW)], mbuf)

    def cm(i, carry):
        idx = mbuf[pl.ds(i * 16, 16)]
        plsc.addupdate_scatter(macc, [idx], ones16)
        return carry
    lax.fori_loop(0, MDW // 16, cm, 0)

    pltpu.sync_copy(acc, cnt_out.at[c, s])
    pltpu.sync_copy(macc, mcnt_out.at[c, s])


_deg_call = pl.kernel(
    _deg_body,
    out_type=(jax.ShapeDtypeStruct((2, 16, NROW), jnp.float32),
              jax.ShapeDtypeStruct((2, 16, 128), jnp.float32)),
    mesh=_mesh,
    scratch_types=[
        pltpu.VMEM((EW,), jnp.int32),
        pltpu.VMEM((NROW,), jnp.float32),
        pltpu.VMEM((MDW,), jnp.int32),
        pltpu.VMEM((128,), jnp.float32),
    ],
    compiler_params=pltpu.CompilerParams(needs_layout_passes=False),
)


def _agg_body(xs_hbm, src_hbm, dst_hbm, out_hbm, idx_s, idx_d, rows, acc,
              sem):
    c = lax.axis_index("c")
    s = lax.axis_index("s")
    wid = c * 16 + s
    zeros16 = jnp.zeros((16,), jnp.float32)

    def zr(i, carry):
        def zc(j, carry2):
            rows[i, pl.ds(j * 16, 16)] = zeros16
            return carry2
        return lax.fori_loop(0, 8, zc, carry)
    lax.fori_loop(0, 128, zr, 0)
    for k in range(5):
        pltpu.sync_copy(rows, acc.at[pl.ds(s * 640 + k * 128, 128)])
    plsc.subcore_barrier()

    def step(k, carry):
        base = wid * EW + k * 128
        pltpu.sync_copy(src_hbm.at[pl.ds(base, 128)], idx_s)
        pltpu.sync_copy(dst_hbm.at[pl.ds(base, 128)], idx_d)
        pltpu.async_copy(xs_hbm.at[idx_s], rows, sem).wait()
        pltpu.sync_copy(rows, acc.at[idx_d], add=True)
        return carry
    lax.fori_loop(0, ECH, step, 0)

    plsc.subcore_barrier()
    for k in range(5):
        pltpu.sync_copy(acc.at[pl.ds(s * 640 + k * 128, 128)],
                        out_hbm.at[c, s, k])


_agg_call = pl.kernel(
    _agg_body,
    out_type=jax.ShapeDtypeStruct((2, 16, 5, 128, FD), jnp.float32),
    mesh=_mesh,
    scratch_types=[
        pltpu.VMEM((128,), jnp.int32),
        pltpu.VMEM((128,), jnp.int32),
        pltpu.VMEM((128, FD), jnp.float32),
        pltpu.VMEM_SHARED((NROW, FD), jnp.float32),
        pltpu.SemaphoreType.DMA,
    ],
)


def _tca_body(x_ref, cnt_ref, wl_ref, bl_ref, wc0_ref,
              xs1_ref, xw1_ref, dinv_ref):
    cnt = jnp.sum(cnt_ref[...], axis=0)
    dinv = lax.rsqrt(cnt + 1.0)
    h = _lrelu(_dot(x_ref[...], wl_ref[...]) + bl_ref[...][None, :])
    xw1 = _dot(h, wc0_ref[...])
    xs1_ref[...] = xw1 * dinv[:, None]
    xw1_ref[...] = xw1
    dinv_ref[...] = dinv


def _tca(xp, cnt32, W_lin, b_lin, Wc0):
    return pl.pallas_call(
        _tca_body,
        grid=(NBLK,),
        in_specs=[
            pl.BlockSpec((RB, FD), lambda i: (i, 0)),
            pl.BlockSpec((32, RB), lambda i: (0, i)),
            pl.BlockSpec((FD, FD), lambda i: (0, 0)),
            pl.BlockSpec((FD,), lambda i: (0,)),
            pl.BlockSpec((FD, FD), lambda i: (0, 0)),
        ],
        out_specs=[
            pl.BlockSpec((RB, FD), lambda i: (i, 0)),
            pl.BlockSpec((RB, FD), lambda i: (i, 0)),
            pl.BlockSpec((RB,), lambda i: (i,)),
        ],
        out_shape=[
            jax.ShapeDtypeStruct((NROW, FD), jnp.float32),
            jax.ShapeDtypeStruct((NROW, FD), jnp.float32),
            jax.ShapeDtypeStruct((NROW,), jnp.float32),
        ],
    )(xp, cnt32, W_lin, b_lin, Wc0)


def _tcc_body(p0_ref, p1_ref, xw1_ref, dinv_ref, bc0_ref, wc1_ref,
              xs2_ref, xw2_ref):
    dinv = dinv_ref[...]
    agg = p0_ref[...] + p1_ref[...]
    out1 = _lrelu(agg * dinv[:, None] + xw1_ref[...] * (dinv * dinv)[:, None]
                  + bc0_ref[...][None, :])
    xw2 = _dot(out1, wc1_ref[...])
    xs2_ref[...] = xw2 * dinv[:, None]
    xw2_ref[...] = xw2


def _tcc(p0, p1, xw1, dinv, bc0, Wc1):
    return pl.pallas_call(
        _tcc_body,
        grid=(NBLK,),
        in_specs=[
            pl.BlockSpec((RB, FD), lambda i: (i, 0)),
            pl.BlockSpec((RB, FD), lambda i: (i, 0)),
            pl.BlockSpec((RB, FD), lambda i: (i, 0)),
            pl.BlockSpec((RB,), lambda i: (i,)),
            pl.BlockSpec((FD,), lambda i: (0,)),
            pl.BlockSpec((FD, FD), lambda i: (0, 0)),
        ],
        out_specs=[
            pl.BlockSpec((RB, FD), lambda i: (i, 0)),
            pl.BlockSpec((RB, FD), lambda i: (i, 0)),
        ],
        out_shape=[
            jax.ShapeDtypeStruct((NROW, FD), jnp.float32),
            jax.ShapeDtypeStruct((NROW, FD), jnp.float32),
        ],
    )(p0, p1, xw1, dinv, bc0, Wc1)


def _te1_body(p0_ref, p1_ref, xw2_ref, dinv_ref, bc1_ref, wg_ref, md_ref,
              bg_ref, wcls_ref, bcls_ref, ls_ref, seg_ref):
    i = pl.program_id(0)
    dinv = dinv_ref[...]
    h2 = _lrelu((p0_ref[...] + p1_ref[...]) * dinv[:, None]
                + xw2_ref[...] * (dinv * dinv)[:, None]
                + bc1_ref[...][None, :])
    xwm = _dot(h2, wg_ref[...])
    ob = _lrelu(xwm + bg_ref[...][None, :])
    logits = _dot(ob, wcls_ref[...]) + bcls_ref[...][None, :]
    ls_ref[...] = _logsoftmax(logits)

    iota = lax.broadcasted_iota(jnp.int32, (128, RB), 0)
    oh = jnp.where(iota == md_ref[...][None, :], 1.0, 0.0).astype(jnp.float32)
    segc = _dot(oh, xwm)

    @pl.when(i == 0)
    def _init():
        seg_ref[...] = segc

    @pl.when(i != 0)
    def _accum():
        seg_ref[...] = seg_ref[...] + segc


def _te1(p0, p1, xw2, dinv, bc1, Wg, mdp, bg, Wcls, bcls):
    return pl.pallas_call(
        _te1_body,
        grid=(NBLK,),
        in_specs=[
            pl.BlockSpec((RB, FD), lambda i: (i, 0)),
            pl.BlockSpec((RB, FD), lambda i: (i, 0)),
            pl.BlockSpec((RB, FD), lambda i: (i, 0)),
            pl.BlockSpec((RB,), lambda i: (i,)),
            pl.BlockSpec((FD,), lambda i: (0,)),
            pl.BlockSpec((FD, FD), lambda i: (0, 0)),
            pl.BlockSpec((RB,), lambda i: (i,)),
            pl.BlockSpec((FD,), lambda i: (0,)),
            pl.BlockSpec((FD, NCLS), lambda i: (0, 0)),
            pl.BlockSpec((NCLS,), lambda i: (0,)),
        ],
        out_specs=[
            pl.BlockSpec((RB, NCLS), lambda i: (i, 0)),
            pl.BlockSpec((128, FD), lambda i: (0, 0)),
        ],
        out_shape=[
            jax.ShapeDtypeStruct((NROW, NCLS), jnp.float32),
            jax.ShapeDtypeStruct((128, FD), jnp.float32),
        ],
    )(p0, p1, xw2, dinv, bc1, Wg, mdp, bg, Wcls, bcls)


def _te2_body(seg_ref, mcnt_ref, mx_ref, wm_ref, bm_ref, wg_ref, bg_ref,
              wcls_ref, bcls_ref, out_ref):
    mcnt = jnp.sum(mcnt_ref[...], axis=0)
    dinvm = lax.rsqrt(mcnt + 1.0)
    mh = _lrelu(_dot(mx_ref[...], wm_ref[...]) + bm_ref[...][None, :])
    xwm = _dot(mh, wg_ref[...])
    seg104 = seg_ref[0:104, :]
    d104 = dinvm[0:104]
    hm = _lrelu(seg104 * d104[:, None] + xwm * (d104 * d104)[:, None]
                + bg_ref[...][None, :])
    logits = _dot(hm, wcls_ref[...]) + bcls_ref[...][None, :]
    out_ref[...] = _logsoftmax(logits)


def _te2(seg, mcnt32, mxp, W_meta, b_meta, Wg, bg, Wcls, bcls):
    return pl.pallas_call(
        _te2_body,
        out_shape=jax.ShapeDtypeStruct((104, NCLS), jnp.float32),
    )(seg, mcnt32, mxp, W_meta, b_meta, Wg, bg, Wcls, bcls)


def kernel(x, meta_x, edge_index, meta_edge_index, W_lin, b_lin, W_meta,
           b_meta, Wc0, bc0, Wc1, bc1, Wg, bg, Wcls, bcls):
    src = edge_index[0].astype(jnp.int32)
    dst = edge_index[1].astype(jnp.int32)
    srcp = jnp.concatenate([src, jnp.zeros((EPAD - EE,), jnp.int32)])
    dstp = jnp.concatenate([dst, jnp.full((EPAD - EE,), NN, jnp.int32)])
    mdp = jnp.concatenate([meta_edge_index[1].astype(jnp.int32) - NN,
                           jnp.full((MDPAD - NN,), MM, jnp.int32)])
    xp = jnp.concatenate([x, jnp.zeros((NROW - NN, FD), x.dtype)], axis=0)
    mxp = jnp.concatenate([meta_x, jnp.zeros((104 - MM, FD), meta_x.dtype)],
                          axis=0)

    cnt_parts, mcnt_parts = _deg_call(dstp, mdp)
    cnt32 = cnt_parts.reshape(32, NROW)
    mcnt32 = mcnt_parts.reshape(32, 128)

    xs1, xw1, dinv = _tca(xp, cnt32, W_lin, b_lin, Wc0)
    agg1 = _agg_call(xs1, srcp, dstp).reshape(2, NROW, FD)
    xs2, xw2 = _tcc(agg1[0], agg1[1], xw1, dinv, bc0, Wc1)
    agg2 = _agg_call(xs2, srcp, dstp).reshape(2, NROW, FD)
    base_ls, seg = _te1(agg2[0], agg2[1], xw2, dinv, bc1, Wg, mdp, bg,
                        Wcls, bcls)
    meta_ls = _te2(seg, mcnt32, mxp, W_meta, b_meta, Wg, bg, Wcls, bcls)
    return jnp.concatenate([base_ls[:NN], meta_ls[:MM]], axis=0)
